# Initial kernel scaffold; baseline (speedup 1.0000x reference)
import jax
import jax.numpy as jnp
from jax.experimental import pallas as pl


def kernel(x, edge_index, W):
    raise NotImplementedError("write your pallas kernel here")


# speedup vs baseline: 7.5931x; 7.5931x over previous
import functools

import jax
import jax.numpy as jnp
from jax import lax
from jax.experimental import pallas as pl
from jax.experimental.pallas import tpu as pltpu
from jax.experimental.pallas import tpu_sc as plsc

N = 10000
D = 128
NC = 2
NS = 16
E = 320000
CK = 128
CH = (E + NC * NS * CK - 1) // (NC * NS * CK)
E_PAD = NC * NS * CK * CH
N2 = 10112
RPT = N2 // NS


def _matmul(x_pad, W):
    def body(x_ref, w_ref, out_ref):
        out_ref[...] = jax.lax.dot_general(
            x_ref[...], w_ref[...], (((1,), (1,)), ((), ())),
            preferred_element_type=jnp.float32)

    return pl.pallas_call(
        body,
        grid=(NS,),
        in_specs=[
            pl.BlockSpec((RPT, D), lambda i: (i, 0)),
            pl.BlockSpec((D, D), lambda i: (0, 0)),
        ],
        out_specs=pl.BlockSpec((RPT, D), lambda i: (i, 0)),
        out_shape=jax.ShapeDtypeStruct((N2, D), jnp.float32),
    )(x_pad, W)


def _combine(p0, p1, h):
    def body(a_ref, b_ref, h_ref, out_ref):
        out_ref[...] = a_ref[...] + b_ref[...] - h_ref[...]

    spec = pl.BlockSpec((1000, D), lambda i: (i, 0))
    return pl.pallas_call(
        body,
        grid=(10,),
        in_specs=[spec, spec, spec],
        out_specs=spec,
        out_shape=jax.ShapeDtypeStruct((N, D), jnp.float32),
    )(p0, p1, h)


def _make_scatter():
    mesh = plsc.VectorSubcoreMesh(core_axis_name="c", subcore_axis_name="s")

    @functools.partial(
        pl.kernel,
        out_type=(jax.ShapeDtypeStruct((N2, D), jnp.float32),
                  jax.ShapeDtypeStruct((N2, D), jnp.float32)),
        mesh=mesh,
        scratch_types=[
            pltpu.VMEM((CH, CK), jnp.int32),
            pltpu.VMEM((CH, CK), jnp.int32),
            pltpu.VMEM((CK, D), jnp.float32),
            pltpu.VMEM_SHARED((N2, D), jnp.float32),
            pltpu.SemaphoreType.DMA,
        ],
    )
    def scatter(h_hbm, src_hbm, dst_hbm, p0_hbm, p1_hbm, src_v, dst_v, rows_v,
                acc, sem):
        c = lax.axis_index("c")
        s = lax.axis_index("s")
        r0 = s * RPT
        pltpu.sync_copy(src_hbm.at[c, s], src_v)
        pltpu.sync_copy(dst_hbm.at[c, s], dst_v)
        pltpu.sync_copy(h_hbm.at[pl.ds(r0, RPT)], acc.at[pl.ds(r0, RPT)])
        plsc.subcore_barrier()

        def body(j, carry):
            pltpu.async_copy(h_hbm.at[src_v.at[j]], rows_v, sem).wait()
            pltpu.sync_copy(rows_v, acc.at[dst_v.at[j]], add=True)
            return carry

        lax.fori_loop(0, CH, body, 0)
        plsc.subcore_barrier()

        @pl.when(c == 0)
        def _():
            pltpu.sync_copy(acc.at[pl.ds(r0, RPT)], p0_hbm.at[pl.ds(r0, RPT)])

        @pl.when(c == 1)
        def _():
            pltpu.sync_copy(acc.at[pl.ds(r0, RPT)], p1_hbm.at[pl.ds(r0, RPT)])

    return scatter


_scatter = _make_scatter()


def kernel(x, edge_index, W):
    ei = edge_index.astype(jnp.int32)
    dst = jnp.concatenate([ei[0], jnp.full((E_PAD - E,), N, jnp.int32)])
    src = jnp.concatenate([ei[1], jnp.zeros((E_PAD - E,), jnp.int32)])
    x_pad = jnp.pad(x, ((0, N2 - N), (0, 0)))
    h = _matmul(x_pad, W)
    p0, p1 = _scatter(h, src.reshape(NC, NS, CH, CK), dst.reshape(NC, NS, CH, CK))
    return _combine(p0, p1, h)
